# baseline (speedup 1.0000x reference)
import functools

import jax
import jax.numpy as jnp
from jax import lax
from jax.experimental import pallas as pl
from jax.experimental.pallas import tpu as pltpu
from jax.experimental.pallas import tpu_sc as plsc

_D = 128
_H = 512
_NC = 2
_NS = 16
_CH = 80


def _mlp3_body(gath_ref, pred_ref, w1_ref, b1_ref, w2_ref, b2_ref,
               outp_ref, outs_ref, outo_ref):
    x = jnp.concatenate([gath_ref[...], pred_ref[...]], axis=1)
    x = x.astype(jnp.bfloat16)
    h = jnp.dot(x, w1_ref[...].astype(jnp.bfloat16),
                preferred_element_type=jnp.float32)
    h = jnp.maximum(h + b1_ref[...], 0.0)
    hb = h.astype(jnp.bfloat16)
    w2 = w2_ref[...].astype(jnp.bfloat16)
    ygp = jnp.dot(hb[:, 0 * _H:1 * _H], w2[0], preferred_element_type=jnp.float32)
    ygs = jnp.dot(hb[:, 1 * _H:2 * _H], w2[1], preferred_element_type=jnp.float32)
    ygo = jnp.dot(hb[:, 2 * _H:3 * _H], w2[2], preferred_element_type=jnp.float32)
    outp_ref[...] = ygp + b2_ref[:, 0 * _D:1 * _D]
    outs_ref[...] = ygs + b2_ref[:, 1 * _D:2 * _D]
    outo_ref[...] = ygo + b2_ref[:, 2 * _D:3 * _D]


def _mlp3(gathered, pred_vectors, w1cat, b1cat, w2stack, b2cat, block=2560):
    n = pred_vectors.shape[0]
    grid = (n // block,)
    row_spec = pl.BlockSpec((block, _D), lambda i: (i, 0))
    full = lambda s: pl.BlockSpec(s, lambda i: tuple(0 for _ in s))
    out_shape = [jax.ShapeDtypeStruct((n, _D), jnp.float32)] * 3
    return pl.pallas_call(
        _mlp3_body,
        grid=grid,
        in_specs=[
            row_spec,
            row_spec,
            full((2 * _D, 3 * _H)),
            full((1, 3 * _H)),
            full((3, _H, _D)),
            full((1, 3 * _D)),
        ],
        out_specs=[row_spec, row_spec, row_spec],
        out_shape=out_shape,
    )(gathered, pred_vectors, w1cat, b1cat, w2stack, b2cat)


def _make_scatter(n_pred, n_obj):
    rows_pt = n_pred // _NS
    nblocks = rows_pt // _CH
    assert rows_pt % _CH == 0
    mesh = plsc.VectorSubcoreMesh(core_axis_name="c", subcore_axis_name="s")

    @functools.partial(
        pl.kernel,
        out_type=jax.ShapeDtypeStruct((_NC, n_obj, _D), jnp.float32),
        mesh=mesh,
        scratch_types=[
            pltpu.VMEM_SHARED((n_obj, _D), jnp.float32),
            pltpu.VMEM((_CH, _D), jnp.float32),
            pltpu.VMEM((_CH, _D), jnp.float32),
            pltpu.VMEM((_CH,), jnp.int32),
            pltpu.VMEM((_CH,), jnp.int32),
            pltpu.VMEM((_CH, _D), jnp.float32),
            pltpu.SemaphoreType.DMA,
            pltpu.SemaphoreType.DMA,
            pltpu.SemaphoreType.DMA,
            pltpu.SemaphoreType.DMA,
        ],
    )
    def scatter_kernel(ygs_hbm, ygo_hbm, uidx_hbm, zacc_hbm, ones_hbm,
                       out_hbm, acc_sh, bufa, bufb, idxa, idxb, ones_v,
                       sema, semb, isema, isemb):
        c = lax.axis_index("c")
        s = lax.axis_index("s")

        @pl.when(s == 0)
        def _():
            pltpu.sync_copy(zacc_hbm, acc_sh)

        pltpu.sync_copy(ones_hbm, ones_v)
        plsc.subcore_barrier()
        base = s * rows_pt
        nb2 = nblocks // 2

        def run_values(y_hbm, coff):
            def issue(blk, buf, ibuf, sem, isem):
                pltpu.async_copy(y_hbm.at[pl.ds(base + blk * _CH, _CH)], buf, sem)
                pltpu.async_copy(uidx_hbm.at[s, coff + blk], ibuf, isem)

            def wait(buf, ibuf, sem, isem):
                pltpu.make_async_copy(y_hbm.at[pl.ds(base, _CH)], buf, sem).wait()
                pltpu.make_async_copy(uidx_hbm.at[s, coff], ibuf, isem).wait()

            issue(0, bufa, idxa, sema, isema)
            issue(1, bufb, idxb, semb, isemb)

            def body(b2, carry):
                wait(bufa, idxa, sema, isema)
                pltpu.sync_copy(bufa, acc_sh.at[idxa], add=True)

                @pl.when(b2 < nb2 - 1)
                def _():
                    issue(2 * b2 + 2, bufa, idxa, sema, isema)

                wait(bufb, idxb, semb, isemb)
                pltpu.sync_copy(bufb, acc_sh.at[idxb], add=True)

                @pl.when(b2 < nb2 - 1)
                def _():
                    issue(2 * b2 + 3, bufb, idxb, semb, isemb)

                return carry

            lax.fori_loop(0, nb2, body, 0)

        def run_counts(coff):
            def issue(blk, ibuf, isem):
                pltpu.async_copy(uidx_hbm.at[s, coff + blk], ibuf, isem)

            def wait(ibuf, isem):
                pltpu.make_async_copy(uidx_hbm.at[s, coff], ibuf, isem).wait()

            issue(0, idxa, isema)
            issue(1, idxb, isemb)

            def body(b2, carry):
                wait(idxa, isema)
                pltpu.sync_copy(ones_v, acc_sh.at[idxa], add=True)

                @pl.when(b2 < nb2 - 1)
                def _():
                    issue(2 * b2 + 2, idxa, isema)

                wait(idxb, isemb)
                pltpu.sync_copy(ones_v, acc_sh.at[idxb], add=True)

                @pl.when(b2 < nb2 - 1)
                def _():
                    issue(2 * b2 + 3, idxb, isemb)

                return carry

            lax.fori_loop(0, nb2, body, 0)

        @pl.when(c == 0)
        def _():
            run_values(ygs_hbm, 0)
            run_values(ygo_hbm, nblocks)

        @pl.when(c == 1)
        def _():
            run_counts(0)
            run_counts(nblocks)

        plsc.subcore_barrier()
        rpw8 = (n_obj // _NS) // 8 * 8
        r0 = s * rpw8
        pltpu.sync_copy(acc_sh.at[pl.ds(r0, rpw8)], out_hbm.at[c, pl.ds(r0, rpw8)])
        rem = n_obj - _NS * rpw8
        if rem:
            @pl.when(s == 0)
            def _():
                pltpu.sync_copy(acc_sh.at[pl.ds(_NS * rpw8, rem)],
                                out_hbm.at[c, pl.ds(_NS * rpw8, rem)])

    return scatter_kernel


def _finalize_body(acc_ref, out_ref):
    out_ref[...] = acc_ref[0] / jnp.maximum(acc_ref[1], 1.0)


def _finalize(acc, block=2000):
    n = acc.shape[1]
    return pl.pallas_call(
        _finalize_body,
        grid=(n // block,),
        in_specs=[pl.BlockSpec((_NC, block, _D), lambda i: (0, i, 0))],
        out_specs=pl.BlockSpec((block, _D), lambda i: (i, 0)),
        out_shape=jax.ShapeDtypeStruct((n, _D), jnp.float32),
    )(acc)


def kernel(obj_vectors, pred_vectors, gp_W1, gp_b1, gp_W2, gp_b2,
           gs_W1, gs_b1, gs_W2, gs_b2, go_W1, go_b1, go_W2, go_b2, edges):
    n_obj = obj_vectors.shape[0]
    n_pred = pred_vectors.shape[0]
    s_idx = edges[:, 0]
    o_idx = edges[:, 1]

    def fold(w1):
        return jnp.concatenate([w1[:_D] + w1[2 * _D:], w1[_D:2 * _D]], axis=0)

    w1cat = jnp.concatenate([fold(gp_W1), fold(gs_W1), fold(go_W1)], axis=1)
    b1cat = jnp.concatenate([gp_b1, gs_b1, go_b1])[None, :]
    w2stack = jnp.stack([gp_W2, gs_W2, go_W2])
    b2cat = jnp.concatenate([gp_b2, gs_b2, go_b2])[None, :]

    gathered = jnp.take(obj_vectors, o_idx, axis=0)
    new_pred, ygs, ygo = _mlp3(gathered, pred_vectors, w1cat, b1cat,
                               w2stack, b2cat)

    uidx = jnp.concatenate([s_idx.reshape(_NS, -1, _CH),
                            o_idx.reshape(_NS, -1, _CH)], axis=1)
    zacc = jnp.zeros((n_obj, _D), jnp.float32)
    ones = jnp.ones((_CH, _D), jnp.float32)

    acc = _make_scatter(n_pred, n_obj)(ygs, ygo, uidx, zacc, ones)
    new_obj = _finalize(acc)
    return (new_obj, new_pred)


# speedup vs baseline: 1.7542x; 1.7542x over previous
import functools

import jax
import jax.numpy as jnp
from jax import lax
from jax.experimental import pallas as pl
from jax.experimental.pallas import tpu as pltpu
from jax.experimental.pallas import tpu_sc as plsc

_D = 128
_H = 512
_NC = 2
_NS = 16
_CH = 80


def _mlp3_body(gath_ref, pred_ref, w1_ref, b1_ref, w2_ref, b2_ref,
               outp_ref, outs_ref, outo_ref):
    x = jnp.concatenate([gath_ref[...], pred_ref[...]], axis=1)
    h = jnp.dot(x, w1_ref[...], preferred_element_type=jnp.float32)
    h = jnp.maximum(h + b1_ref[...], 0.0)
    ygp = jnp.dot(h[:, 0 * _H:1 * _H], w2_ref[0], preferred_element_type=jnp.float32)
    ygs = jnp.dot(h[:, 1 * _H:2 * _H], w2_ref[1], preferred_element_type=jnp.float32)
    ygo = jnp.dot(h[:, 2 * _H:3 * _H], w2_ref[2], preferred_element_type=jnp.float32)
    outp_ref[...] = ygp + b2_ref[:, 0 * _D:1 * _D]
    outs_ref[...] = ygs + b2_ref[:, 1 * _D:2 * _D]
    outo_ref[...] = ygo + b2_ref[:, 2 * _D:3 * _D]


def _mlp3(gathered, pred_vectors, w1cat, b1cat, w2stack, b2cat, block=2560):
    n = pred_vectors.shape[0]
    grid = (n // block,)
    row_spec = pl.BlockSpec((block, _D), lambda i: (i, 0))
    full = lambda s: pl.BlockSpec(s, lambda i: tuple(0 for _ in s))
    out_shape = [jax.ShapeDtypeStruct((n, _D), jnp.float32)] * 3
    return pl.pallas_call(
        _mlp3_body,
        grid=grid,
        in_specs=[
            row_spec,
            row_spec,
            full((2 * _D, 3 * _H)),
            full((1, 3 * _H)),
            full((3, _H, _D)),
            full((1, 3 * _D)),
        ],
        out_specs=[row_spec, row_spec, row_spec],
        out_shape=out_shape,
    )(gathered, pred_vectors, w1cat, b1cat, w2stack, b2cat)


def _make_gather(n_pred, n_obj):
    rows_pt = n_pred // (_NC * _NS)
    nchunks = rows_pt // _CH
    assert rows_pt % _CH == 0
    mesh = plsc.VectorSubcoreMesh(core_axis_name="c", subcore_axis_name="s")

    @functools.partial(
        pl.kernel,
        out_type=jax.ShapeDtypeStruct((n_pred, _D), jnp.float32),
        mesh=mesh,
        scratch_types=[
            pltpu.VMEM_SHARED((n_obj, _D), jnp.float32),
            pltpu.VMEM((_CH, _D), jnp.float32),
            pltpu.VMEM((_CH, _D), jnp.float32),
            pltpu.VMEM((_CH,), jnp.int32),
            pltpu.VMEM((_CH,), jnp.int32),
            pltpu.SemaphoreType.DMA,
            pltpu.SemaphoreType.DMA,
            pltpu.SemaphoreType.DMA,
            pltpu.SemaphoreType.DMA,
        ],
    )
    def gather_kernel(obj_hbm, gidx_hbm, out_hbm, obj_sh, bufa, bufb, ia, ib,
                      oa, ob, isa, isb):
        c = lax.axis_index("c")
        s = lax.axis_index("s")
        wid = s * _NC + c

        @pl.when(s == 0)
        def _():
            pltpu.sync_copy(obj_hbm, obj_sh)

        plsc.subcore_barrier()
        base = wid * rows_pt
        nb2 = (nchunks + 1) // 2

        def issue_idx(blk, ibuf, isem):
            pltpu.async_copy(gidx_hbm.at[wid, blk], ibuf, isem)

        def wait_idx(ibuf, isem):
            pltpu.make_async_copy(gidx_hbm.at[wid, 0], ibuf, isem).wait()

        def issue_out(blk, buf, osem):
            pltpu.async_copy(buf, out_hbm.at[pl.ds(base + blk * _CH, _CH)], osem)

        def wait_out(buf, osem):
            pltpu.make_async_copy(buf, out_hbm.at[pl.ds(base, _CH)], osem).wait()

        issue_idx(0, ia, isa)
        issue_idx(1, ib, isb)

        def body(b2, carry):
            wait_idx(ia, isa)

            @pl.when(b2 > 0)
            def _():
                wait_out(bufa, oa)

            pltpu.sync_copy(obj_sh.at[ia], bufa)
            issue_out(2 * b2, bufa, oa)

            @pl.when(b2 < nb2 - 1)
            def _():
                issue_idx(2 * b2 + 2, ia, isa)

            @pl.when(2 * b2 + 1 < nchunks)
            def _():
                wait_idx(ib, isb)

                @pl.when(b2 > 0)
                def _():
                    wait_out(bufb, ob)

                pltpu.sync_copy(obj_sh.at[ib], bufb)
                issue_out(2 * b2 + 1, bufb, ob)

                @pl.when(2 * b2 + 3 < nchunks)
                def _():
                    issue_idx(2 * b2 + 3, ib, isb)

            return carry

        lax.fori_loop(0, nb2, body, 0)
        wait_out(bufa, oa)
        wait_out(bufb, ob)

    return gather_kernel


def _make_scatter(n_pred, n_obj):
    rows_pt = n_pred // _NS
    nblocks = rows_pt // _CH
    assert rows_pt % _CH == 0
    mesh = plsc.VectorSubcoreMesh(core_axis_name="c", subcore_axis_name="s")

    @functools.partial(
        pl.kernel,
        out_type=jax.ShapeDtypeStruct((_NC, n_obj, _D), jnp.float32),
        mesh=mesh,
        scratch_types=[
            pltpu.VMEM_SHARED((n_obj, _D), jnp.float32),
            pltpu.VMEM((_CH, _D), jnp.float32),
            pltpu.VMEM((_CH, _D), jnp.float32),
            pltpu.VMEM((_CH,), jnp.int32),
            pltpu.VMEM((_CH,), jnp.int32),
            pltpu.VMEM((_CH, _D), jnp.float32),
            pltpu.SemaphoreType.DMA,
            pltpu.SemaphoreType.DMA,
            pltpu.SemaphoreType.DMA,
            pltpu.SemaphoreType.DMA,
        ],
    )
    def scatter_kernel(ygs_hbm, ygo_hbm, uidx_hbm, zacc_hbm, ones_hbm,
                       out_hbm, acc_sh, bufa, bufb, idxa, idxb, ones_v,
                       sema, semb, isema, isemb):
        c = lax.axis_index("c")
        s = lax.axis_index("s")

        @pl.when(s == 0)
        def _():
            pltpu.sync_copy(zacc_hbm, acc_sh)

        pltpu.sync_copy(ones_hbm, ones_v)
        plsc.subcore_barrier()
        base = s * rows_pt
        nb2 = nblocks // 2

        def run_values(y_hbm, coff):
            def issue(blk, buf, ibuf, sem, isem):
                pltpu.async_copy(y_hbm.at[pl.ds(base + blk * _CH, _CH)], buf, sem)
                pltpu.async_copy(uidx_hbm.at[s, coff + blk], ibuf, isem)

            def wait(buf, ibuf, sem, isem):
                pltpu.make_async_copy(y_hbm.at[pl.ds(base, _CH)], buf, sem).wait()
                pltpu.make_async_copy(uidx_hbm.at[s, coff], ibuf, isem).wait()

            issue(0, bufa, idxa, sema, isema)
            issue(1, bufb, idxb, semb, isemb)

            def body(b2, carry):
                wait(bufa, idxa, sema, isema)
                pltpu.sync_copy(bufa, acc_sh.at[idxa], add=True)

                @pl.when(b2 < nb2 - 1)
                def _():
                    issue(2 * b2 + 2, bufa, idxa, sema, isema)

                wait(bufb, idxb, semb, isemb)
                pltpu.sync_copy(bufb, acc_sh.at[idxb], add=True)

                @pl.when(b2 < nb2 - 1)
                def _():
                    issue(2 * b2 + 3, bufb, idxb, semb, isemb)

                return carry

            lax.fori_loop(0, nb2, body, 0)

        def run_counts(coff):
            def issue(blk, ibuf, isem):
                pltpu.async_copy(uidx_hbm.at[s, coff + blk], ibuf, isem)

            def wait(ibuf, isem):
                pltpu.make_async_copy(uidx_hbm.at[s, coff], ibuf, isem).wait()

            issue(0, idxa, isema)
            issue(1, idxb, isemb)

            def body(b2, carry):
                wait(idxa, isema)
                pltpu.sync_copy(ones_v, acc_sh.at[idxa], add=True)

                @pl.when(b2 < nb2 - 1)
                def _():
                    issue(2 * b2 + 2, idxa, isema)

                wait(idxb, isemb)
                pltpu.sync_copy(ones_v, acc_sh.at[idxb], add=True)

                @pl.when(b2 < nb2 - 1)
                def _():
                    issue(2 * b2 + 3, idxb, isemb)

                return carry

            lax.fori_loop(0, nb2, body, 0)

        @pl.when(c == 0)
        def _():
            run_values(ygs_hbm, 0)
            run_values(ygo_hbm, nblocks)

        @pl.when(c == 1)
        def _():
            run_counts(0)
            run_counts(nblocks)

        plsc.subcore_barrier()
        rpw8 = (n_obj // _NS) // 8 * 8
        r0 = s * rpw8
        pltpu.sync_copy(acc_sh.at[pl.ds(r0, rpw8)], out_hbm.at[c, pl.ds(r0, rpw8)])
        rem = n_obj - _NS * rpw8
        if rem:
            @pl.when(s == 0)
            def _():
                pltpu.sync_copy(acc_sh.at[pl.ds(_NS * rpw8, rem)],
                                out_hbm.at[c, pl.ds(_NS * rpw8, rem)])

    return scatter_kernel


def _finalize_body(acc_ref, out_ref):
    out_ref[...] = acc_ref[0] / jnp.maximum(acc_ref[1], 1.0)


def _finalize(acc, block=2000):
    n = acc.shape[1]
    return pl.pallas_call(
        _finalize_body,
        grid=(n // block,),
        in_specs=[pl.BlockSpec((_NC, block, _D), lambda i: (0, i, 0))],
        out_specs=pl.BlockSpec((block, _D), lambda i: (i, 0)),
        out_shape=jax.ShapeDtypeStruct((n, _D), jnp.float32),
    )(acc)


def kernel(obj_vectors, pred_vectors, gp_W1, gp_b1, gp_W2, gp_b2,
           gs_W1, gs_b1, gs_W2, gs_b2, go_W1, go_b1, go_W2, go_b2, edges):
    n_obj = obj_vectors.shape[0]
    n_pred = pred_vectors.shape[0]
    s_idx = edges[:, 0]
    o_idx = edges[:, 1]

    def fold(w1):
        return jnp.concatenate([w1[:_D] + w1[2 * _D:], w1[_D:2 * _D]], axis=0)

    w1cat = jnp.concatenate([fold(gp_W1), fold(gs_W1), fold(go_W1)], axis=1)
    b1cat = jnp.concatenate([gp_b1, gs_b1, go_b1])[None, :]
    w2stack = jnp.stack([gp_W2, gs_W2, go_W2])
    b2cat = jnp.concatenate([gp_b2, gs_b2, go_b2])[None, :]

    gidx = o_idx.reshape(_NC * _NS, -1, _CH)
    gathered = _make_gather(n_pred, n_obj)(obj_vectors, gidx)
    new_pred, ygs, ygo = _mlp3(gathered, pred_vectors, w1cat, b1cat,
                               w2stack, b2cat)

    uidx = jnp.concatenate([s_idx.reshape(_NS, -1, _CH),
                            o_idx.reshape(_NS, -1, _CH)], axis=1)
    zacc = jnp.zeros((n_obj, _D), jnp.float32)
    ones = jnp.ones((_CH, _D), jnp.float32)

    acc = _make_scatter(n_pred, n_obj)(ygs, ygo, uidx, zacc, ones)
    new_obj = _finalize(acc)
    return (new_obj, new_pred)


# speedup vs baseline: 2.1209x; 1.2091x over previous
import functools

import jax
import jax.numpy as jnp
from jax import lax
from jax.experimental import pallas as pl
from jax.experimental.pallas import tpu as pltpu
from jax.experimental.pallas import tpu_sc as plsc

_D = 128
_H = 512
_NC = 2
_NS = 16
_CH = 80


def _mlp3_body(gath_ref, pred_ref, w1_ref, b1_ref, w2_ref, b2_ref,
               outp_ref, outs_ref, outo_ref):
    x = jnp.concatenate([gath_ref[...], pred_ref[...]], axis=1)
    h = jnp.dot(x, w1_ref[...], preferred_element_type=jnp.float32)
    h = jnp.maximum(h + b1_ref[...], 0.0)
    ygp = jnp.dot(h[:, 0 * _H:1 * _H], w2_ref[0], preferred_element_type=jnp.float32)
    ygs = jnp.dot(h[:, 1 * _H:2 * _H], w2_ref[1], preferred_element_type=jnp.float32)
    ygo = jnp.dot(h[:, 2 * _H:3 * _H], w2_ref[2], preferred_element_type=jnp.float32)
    outp_ref[...] = ygp + b2_ref[:, 0 * _D:1 * _D]
    outs_ref[...] = ygs + b2_ref[:, 1 * _D:2 * _D]
    outo_ref[...] = ygo + b2_ref[:, 2 * _D:3 * _D]


def _mlp3(gathered, pred_vectors, w1cat, b1cat, w2stack, b2cat, block=2560):
    n = pred_vectors.shape[0]
    grid = (n // block,)
    row_spec = pl.BlockSpec((block, _D), lambda i: (i, 0))
    full = lambda s: pl.BlockSpec(s, lambda i: tuple(0 for _ in s))
    out_shape = [jax.ShapeDtypeStruct((n, _D), jnp.float32)] * 3
    return pl.pallas_call(
        _mlp3_body,
        grid=grid,
        in_specs=[
            row_spec,
            row_spec,
            full((2 * _D, 3 * _H)),
            full((1, 3 * _H)),
            full((3, _H, _D)),
            full((1, 3 * _D)),
        ],
        out_specs=[row_spec, row_spec, row_spec],
        out_shape=out_shape,
    )(gathered, pred_vectors, w1cat, b1cat, w2stack, b2cat)


def _make_gather(n_pred, n_obj):
    rows_pt = n_pred // (_NC * _NS)
    nchunks = rows_pt // _CH
    assert rows_pt % _CH == 0
    mesh = plsc.VectorSubcoreMesh(core_axis_name="c", subcore_axis_name="s")

    @functools.partial(
        pl.kernel,
        out_type=jax.ShapeDtypeStruct((n_pred, _D), jnp.float32),
        mesh=mesh,
        scratch_types=[
            pltpu.VMEM_SHARED((n_obj, _D), jnp.float32),
            pltpu.VMEM((_CH, _D), jnp.float32),
            pltpu.VMEM((_CH, _D), jnp.float32),
            pltpu.VMEM((_CH,), jnp.int32),
            pltpu.VMEM((_CH,), jnp.int32),
            pltpu.SemaphoreType.DMA,
            pltpu.SemaphoreType.DMA,
            pltpu.SemaphoreType.DMA,
            pltpu.SemaphoreType.DMA,
        ],
    )
    def gather_kernel(obj_hbm, gidx_hbm, out_hbm, obj_sh, bufa, bufb, ia, ib,
                      oa, ob, isa, isb):
        c = lax.axis_index("c")
        s = lax.axis_index("s")
        wid = s * _NC + c

        @pl.when(s == 0)
        def _():
            pltpu.sync_copy(obj_hbm, obj_sh)

        plsc.subcore_barrier()
        base = wid * rows_pt
        nb2 = (nchunks + 1) // 2

        def issue_idx(blk, ibuf, isem):
            pltpu.async_copy(gidx_hbm.at[wid, blk], ibuf, isem)

        def wait_idx(ibuf, isem):
            pltpu.make_async_copy(gidx_hbm.at[wid, 0], ibuf, isem).wait()

        def issue_out(blk, buf, osem):
            pltpu.async_copy(buf, out_hbm.at[pl.ds(base + blk * _CH, _CH)], osem)

        def wait_out(buf, osem):
            pltpu.make_async_copy(buf, out_hbm.at[pl.ds(base, _CH)], osem).wait()

        issue_idx(0, ia, isa)
        issue_idx(1, ib, isb)

        def body(b2, carry):
            wait_idx(ia, isa)

            @pl.when(b2 > 0)
            def _():
                wait_out(bufa, oa)

            pltpu.sync_copy(obj_sh.at[ia], bufa)
            issue_out(2 * b2, bufa, oa)

            @pl.when(b2 < nb2 - 1)
            def _():
                issue_idx(2 * b2 + 2, ia, isa)

            @pl.when(2 * b2 + 1 < nchunks)
            def _():
                wait_idx(ib, isb)

                @pl.when(b2 > 0)
                def _():
                    wait_out(bufb, ob)

                pltpu.sync_copy(obj_sh.at[ib], bufb)
                issue_out(2 * b2 + 1, bufb, ob)

                @pl.when(2 * b2 + 3 < nchunks)
                def _():
                    issue_idx(2 * b2 + 3, ib, isb)

            return carry

        lax.fori_loop(0, nb2, body, 0)
        wait_out(bufa, oa)
        wait_out(bufb, ob)

    return gather_kernel


def _pipe(nblocks, issue, wait, use):
    issue(0, 0)
    issue(1, 1)
    nb2 = (nblocks + 1) // 2

    def body(b2, carry):
        wait(0)
        use(0)

        @pl.when(2 * b2 + 2 < nblocks)
        def _():
            issue(2 * b2 + 2, 0)

        @pl.when(2 * b2 + 1 < nblocks)
        def _():
            wait(1)
            use(1)

            @pl.when(2 * b2 + 3 < nblocks)
            def _():
                issue(2 * b2 + 3, 1)

        return carry

    lax.fori_loop(0, nb2, body, 0)


def _writeback(acc_sh, out_hbm, c, s, n_obj):
    rpw8 = (n_obj // _NS) // 8 * 8
    r0 = s * rpw8
    pltpu.sync_copy(acc_sh.at[pl.ds(r0, rpw8)], out_hbm.at[c, pl.ds(r0, rpw8)])
    rem = n_obj - _NS * rpw8
    if rem:
        @pl.when(s == 0)
        def _():
            pltpu.sync_copy(acc_sh.at[pl.ds(_NS * rpw8, rem)],
                            out_hbm.at[c, pl.ds(_NS * rpw8, rem)])


def _make_values(n_pred, n_obj):
    rows_pt = n_pred // (_NC * _NS)
    nblocks = rows_pt // _CH
    assert rows_pt % _CH == 0
    mesh = plsc.VectorSubcoreMesh(core_axis_name="c", subcore_axis_name="s")

    @functools.partial(
        pl.kernel,
        out_type=jax.ShapeDtypeStruct((_NC, n_obj, _D), jnp.float32),
        mesh=mesh,
        scratch_types=[
            pltpu.VMEM_SHARED((n_obj, _D), jnp.float32),
            pltpu.VMEM((_CH, _D), jnp.float32),
            pltpu.VMEM((_CH, _D), jnp.float32),
            pltpu.VMEM((_CH,), jnp.int32),
            pltpu.VMEM((_CH,), jnp.int32),
            pltpu.SemaphoreType.DMA,
            pltpu.SemaphoreType.DMA,
            pltpu.SemaphoreType.DMA,
            pltpu.SemaphoreType.DMA,
        ],
    )
    def values_kernel(ygs_hbm, ygo_hbm, vidx_hbm, zacc_hbm,
                      out_hbm, acc_sh, bufa, bufb, idxa, idxb,
                      sema, semb, isema, isemb):
        c = lax.axis_index("c")
        s = lax.axis_index("s")
        wid = s * _NC + c

        @pl.when(s == 0)
        def _():
            pltpu.sync_copy(zacc_hbm, acc_sh)

        plsc.subcore_barrier()
        base = wid * rows_pt
        bufs = (bufa, bufb)
        idxs = (idxa, idxb)
        sems = (sema, semb)
        isems = (isema, isemb)

        def run_array(y_hbm, coff):
            def issue(blk, sl):
                pltpu.async_copy(y_hbm.at[pl.ds(base + blk * _CH, _CH)],
                                 bufs[sl], sems[sl])
                pltpu.async_copy(vidx_hbm.at[wid, coff + blk], idxs[sl], isems[sl])

            def wait(sl):
                pltpu.make_async_copy(y_hbm.at[pl.ds(base, _CH)],
                                      bufs[sl], sems[sl]).wait()
                pltpu.make_async_copy(vidx_hbm.at[wid, coff],
                                      idxs[sl], isems[sl]).wait()

            def use(sl):
                pltpu.sync_copy(bufs[sl], acc_sh.at[idxs[sl]], add=True)

            _pipe(nblocks, issue, wait, use)

        run_array(ygs_hbm, 0)
        run_array(ygo_hbm, nblocks)
        plsc.subcore_barrier()
        _writeback(acc_sh, out_hbm, c, s, n_obj)

    return values_kernel


def _make_counts(n_pred, n_obj):
    rows_pt = n_pred // (_NC * _NS)
    nblocks = rows_pt // _CH
    mesh = plsc.VectorSubcoreMesh(core_axis_name="c", subcore_axis_name="s")

    @functools.partial(
        pl.kernel,
        out_type=jax.ShapeDtypeStruct((_NC, n_obj, _D), jnp.float32),
        mesh=mesh,
        scratch_types=[
            pltpu.VMEM_SHARED((n_obj, _D), jnp.float32),
            pltpu.VMEM((_CH,), jnp.int32),
            pltpu.VMEM((_CH,), jnp.int32),
            pltpu.VMEM((_CH, _D), jnp.float32),
            pltpu.SemaphoreType.DMA,
            pltpu.SemaphoreType.DMA,
        ],
    )
    def counts_kernel(vidx_hbm, zacc_hbm, ones_hbm,
                      out_hbm, acc_sh, idxa, idxb, ones_v, isema, isemb):
        c = lax.axis_index("c")
        s = lax.axis_index("s")
        wid = s * _NC + c

        @pl.when(s == 0)
        def _():
            pltpu.sync_copy(zacc_hbm, acc_sh)

        pltpu.sync_copy(ones_hbm, ones_v)
        plsc.subcore_barrier()
        idxs = (idxa, idxb)
        isems = (isema, isemb)

        def run_array(coff):
            def issue(blk, sl):
                pltpu.async_copy(vidx_hbm.at[wid, coff + blk], idxs[sl], isems[sl])

            def wait(sl):
                pltpu.make_async_copy(vidx_hbm.at[wid, coff],
                                      idxs[sl], isems[sl]).wait()

            def use(sl):
                pltpu.sync_copy(ones_v, acc_sh.at[idxs[sl]], add=True)

            _pipe(nblocks, issue, wait, use)

        run_array(0)
        run_array(nblocks)
        plsc.subcore_barrier()
        _writeback(acc_sh, out_hbm, c, s, n_obj)

    return counts_kernel


def _finalize_body(acc_ref, cnt_ref, out_ref):
    vals = acc_ref[0] + acc_ref[1]
    cnts = cnt_ref[0] + cnt_ref[1]
    out_ref[...] = vals / jnp.maximum(cnts, 1.0)


def _finalize(acc, cnt, block=2000):
    n = acc.shape[1]
    spec = pl.BlockSpec((_NC, block, _D), lambda i: (0, i, 0))
    return pl.pallas_call(
        _finalize_body,
        grid=(n // block,),
        in_specs=[spec, spec],
        out_specs=pl.BlockSpec((block, _D), lambda i: (i, 0)),
        out_shape=jax.ShapeDtypeStruct((n, _D), jnp.float32),
    )(acc, cnt)


def kernel(obj_vectors, pred_vectors, gp_W1, gp_b1, gp_W2, gp_b2,
           gs_W1, gs_b1, gs_W2, gs_b2, go_W1, go_b1, go_W2, go_b2, edges):
    n_obj = obj_vectors.shape[0]
    n_pred = pred_vectors.shape[0]
    s_idx = edges[:, 0]
    o_idx = edges[:, 1]

    def fold(w1):
        return jnp.concatenate([w1[:_D] + w1[2 * _D:], w1[_D:2 * _D]], axis=0)

    w1cat = jnp.concatenate([fold(gp_W1), fold(gs_W1), fold(go_W1)], axis=1)
    b1cat = jnp.concatenate([gp_b1, gs_b1, go_b1])[None, :]
    w2stack = jnp.stack([gp_W2, gs_W2, go_W2])
    b2cat = jnp.concatenate([gp_b2, gs_b2, go_b2])[None, :]

    gidx = o_idx.reshape(_NC * _NS, -1, _CH)
    gathered = _make_gather(n_pred, n_obj)(obj_vectors, gidx)
    new_pred, ygs, ygo = _mlp3(gathered, pred_vectors, w1cat, b1cat,
                               w2stack, b2cat)

    vidx = jnp.concatenate([s_idx.reshape(_NC * _NS, -1, _CH),
                            o_idx.reshape(_NC * _NS, -1, _CH)], axis=1)
    zacc = jnp.zeros((n_obj, _D), jnp.float32)
    ones = jnp.ones((_CH, _D), jnp.float32)

    cnt = _make_counts(n_pred, n_obj)(vidx, zacc, ones)
    acc = _make_values(n_pred, n_obj)(ygs, ygo, vidx, zacc)
    new_obj = _finalize(acc, cnt)
    return (new_obj, new_pred)
